# baseline (speedup 1.0000x reference)
import functools

import jax
import jax.numpy as jnp
from jax import lax
from jax.experimental import pallas as pl
from jax.experimental.pallas import tpu as pltpu
from jax.experimental.pallas import tpu_sc as plsc

B = 16
C = 768
H = 56
W = 56
R = B * C
NC = 2
NS = 16
NW = NC * NS
IMGS_PER_W = R // NW
K = 8
NCHUNK = IMGS_PER_W // K
NBUF = 2

_mesh = plsc.VectorSubcoreMesh(core_axis_name="c", subcore_axis_name="s")


@functools.partial(
    pl.kernel,
    out_type=jax.ShapeDtypeStruct((R, H, W), jnp.float32),
    mesh=_mesh,
    scratch_types=[
        [pltpu.VMEM((K, H, W), jnp.float32)] * NBUF,
        [pltpu.SemaphoreType.DMA] * NBUF,
        [pltpu.SemaphoreType.DMA] * NBUF,
    ],
)
def _reverse_images(in_hbm, out_hbm, bufs, insems, outsems):
    wid = lax.axis_index("s") * NC + lax.axis_index("c")
    base = wid * IMGS_PER_W
    b = base // C
    s_top = 2 * b * C + (C - 1)

    def issue_in(g, i):
        r0 = base + g * K
        s0 = s_top - r0 - (K - 1)
        pltpu.async_copy(in_hbm.at[pl.ds(s0, K)], bufs[i], insems[i])

    def wait_in(i):
        pltpu.make_async_copy(in_hbm.at[pl.ds(0, K)], bufs[i], insems[i]).wait()

    def issue_out(g, i):
        r0 = base + g * K
        for j in range(K):
            pltpu.async_copy(
                bufs[i].at[pl.ds(K - 1 - j, 1)],
                out_hbm.at[pl.ds(r0 + j, 1)],
                outsems[i],
            )

    def wait_out(g, i):
        r0 = base + g * K
        pltpu.make_async_copy(bufs[i], out_hbm.at[pl.ds(r0, K)], outsems[i]).wait()

    issue_in(0, 0)
    wait_in(0)
    issue_out(0, 0)
    issue_in(1, 1)

    @pl.loop(1, NCHUNK - 1, step=NBUF)
    def _ring(g0):
        for i in range(NBUF):
            g = g0 + i
            bi = (1 + i) % NBUF
            wait_in(bi)
            issue_out(g, bi)
            wait_out(g - 1, 1 - bi)
            issue_in(g + 1, 1 - bi)

    bi = (NCHUNK - 1) % NBUF
    wait_in(bi)
    issue_out(NCHUNK - 1, bi)
    wait_out(NCHUNK - 2, 1 - bi)
    wait_out(NCHUNK - 1, bi)


def kernel(input):
    x = input.reshape(R, H, W)
    y = _reverse_images(x)
    return y.reshape(B, C, H, W)


# speedup vs baseline: 1.0066x; 1.0066x over previous
import functools

import jax
import jax.numpy as jnp
from jax import lax
from jax.experimental import pallas as pl
from jax.experimental.pallas import tpu as pltpu
from jax.experimental.pallas import tpu_sc as plsc

B = 16
C = 768
H = 56
W = 56
R = B * C
NC = 2
NS = 16
NW = NC * NS
IMGS_PER_W = R // NW
K = 4
NCHUNK = IMGS_PER_W // K
NBUF = 4

_mesh = plsc.VectorSubcoreMesh(core_axis_name="c", subcore_axis_name="s")


@functools.partial(
    pl.kernel,
    out_type=jax.ShapeDtypeStruct((R, H, W), jnp.float32),
    mesh=_mesh,
    scratch_types=[
        [pltpu.VMEM((K, H, W), jnp.float32)] * NBUF,
        [pltpu.SemaphoreType.DMA] * NBUF,
        [pltpu.SemaphoreType.DMA] * NBUF,
    ],
)
def _reverse_images(in_hbm, out_hbm, bufs, insems, outsems):
    wid = lax.axis_index("s") * NC + lax.axis_index("c")
    base = wid * IMGS_PER_W
    b = base // C
    s_top = 2 * b * C + (C - 1)

    def issue_in(g, i):
        r0 = base + g * K
        for j in range(K):
            pltpu.async_copy(
                in_hbm.at[pl.ds(s_top - r0 - j, 1)],
                bufs[i].at[pl.ds(j, 1)],
                insems[i],
            )

    def wait_in(i):
        pltpu.make_async_copy(in_hbm.at[pl.ds(0, K)], bufs[i], insems[i]).wait()

    def issue_out(g, i):
        r0 = base + g * K
        pltpu.async_copy(bufs[i], out_hbm.at[pl.ds(r0, K)], outsems[i])

    def wait_out(g, i):
        r0 = base + g * K
        pltpu.make_async_copy(bufs[i], out_hbm.at[pl.ds(r0, K)], outsems[i]).wait()

    for g in range(NBUF):
        issue_in(g, g % NBUF)

    for g in range(NBUF):
        wait_in(g % NBUF)
        issue_out(g, g % NBUF)
        if g >= 2:
            wait_out(g - 2, (g + 2) % NBUF)
            issue_in(g + 2, (g + 2) % NBUF)

    @pl.loop(NBUF, NCHUNK - NBUF, step=NBUF)
    def _ring(g0):
        for i in range(NBUF):
            g = g0 + i
            bi = i
            wait_in(bi)
            issue_out(g, bi)
            wait_out(g - 2, (bi + 2) % NBUF)
            issue_in(g + 2, (bi + 2) % NBUF)

    for g in range(NCHUNK - NBUF, NCHUNK):
        bi = g % NBUF
        wait_in(bi)
        issue_out(g, bi)
        wait_out(g - 2, (bi + 2) % NBUF)
        if g + 2 < NCHUNK:
            issue_in(g + 2, (bi + 2) % NBUF)

    wait_out(NCHUNK - 2, (NCHUNK - 2) % NBUF)
    wait_out(NCHUNK - 1, (NCHUNK - 1) % NBUF)


def kernel(input):
    x = input.reshape(R, H, W)
    y = _reverse_images(x)
    return y.reshape(B, C, H, W)
